# baseline (speedup 1.0000x reference)
import jax
import jax.numpy as jnp
from jax import lax
from jax.experimental import pallas as pl
from jax.experimental.pallas import tpu as pltpu

_N = 8192
_M = 65536
_F = 21
_BQ = 512
_S = 1024


def _win_body(bias_ref, start_ref, posyt_ref, keys_ref, x_ref,
              o0_ref, o1_ref, o2_ref, flag_ref):
    start = start_ref[pl.program_id(0)]
    bias = bias_ref[0]
    big = jnp.float32(jnp.inf)
    qlat = posyt_ref[0:1, :]
    qlon = posyt_ref[1:2, :]
    kl = keys_ref[pl.ds(start, _S), 0:1]
    kn = keys_ref[pl.ds(start, _S), 1:2]
    dlat = kl - qlat
    dlon = kn - qlon
    d2 = dlat * dlat + dlon * dlon

    v1 = jnp.min(d2, axis=0, keepdims=True)
    v2 = jnp.min(jnp.where(d2 > v1, d2, big), axis=0, keepdims=True)
    v3 = jnp.min(jnp.where(d2 > v2, d2, big), axis=0, keepdims=True)
    w_mat = jnp.where(d2 <= v3,
                      1.0 / jnp.maximum(d2 + bias, 1e-16), 0.0)

    num = jax.lax.dot_general(
        w_mat, x_ref[pl.ds(start, _S), :],
        dimension_numbers=(((0,), (0,)), ((), ())),
        preferred_element_type=jnp.float32,
        precision=jax.lax.Precision.DEFAULT)
    fd = _F // 3
    rden = 1.0 / num[:, _F:_F + 1]
    o0_ref[...] = num[:, 0:fd] * rden
    o1_ref[...] = num[:, fd:2 * fd] * rden
    o2_ref[...] = num[:, 2 * fd:3 * fd] * rden

    lat_first = jnp.min(keys_ref[pl.ds(start, 8), 0:1], axis=0,
                        keepdims=True)
    lat_last = jnp.max(keys_ref[pl.ds(start + (_S - 8), 8), 0:1], axis=0,
                       keepdims=True)
    dl = jnp.maximum(qlat - lat_first, 0.0)
    dr = jnp.maximum(lat_last - qlat, 0.0)
    bl = jnp.where(start == 0, big, dl * dl)
    br = jnp.where(start + _S == keys_ref.shape[0], big, dr * dr)
    ok = v3 < jnp.minimum(bl, br)
    flag_ref[...] = ok.astype(jnp.float32).reshape(1, 1, _BQ)


def _full_body(bias_ref, posy_ref, keys_ref, x_ref, out_ref):
    qlat = posy_ref[:, 0:1]
    qlon = posy_ref[:, 1:2]
    klat = keys_ref[0:1, :]
    klon = keys_ref[1:2, :]
    dlat = qlat - klat
    dlon = qlon - klon
    d2 = dlat * dlat + dlon * dlon

    bias = bias_ref[0]
    big = jnp.float32(jnp.inf)
    v1 = jnp.min(d2, axis=1, keepdims=True)
    v2 = jnp.min(jnp.where(d2 > v1, d2, big), axis=1, keepdims=True)
    v3 = jnp.min(jnp.where(d2 > v2, d2, big), axis=1, keepdims=True)
    w_mat = jnp.where(d2 <= v3,
                      1.0 / jnp.maximum(d2 + bias, 1e-16), 0.0)
    den = (1.0 / jnp.maximum(v1 + bias, 1e-16)
           + 1.0 / jnp.maximum(v2 + bias, 1e-16)
           + 1.0 / jnp.maximum(v3 + bias, 1e-16))
    num = jax.lax.dot_general(
        w_mat, x_ref[...],
        dimension_numbers=(((1,), (0,)), ((), ())),
        preferred_element_type=jnp.float32,
        precision=jax.lax.Precision.DEFAULT)
    out_ref[...] = num * (1.0 / den)


def _full_kernel(bias, x, pos_x, pos_y):
    m, n, f = pos_y.shape[0], pos_x.shape[0], x.shape[1]
    return pl.pallas_call(
        _full_body,
        grid=(m // _BQ,),
        in_specs=[
            pl.BlockSpec(memory_space=pltpu.SMEM),
            pl.BlockSpec((_BQ, 2), lambda i: (i, 0)),
            pl.BlockSpec((2, n), lambda i: (0, 0)),
            pl.BlockSpec((n, f), lambda i: (0, 0)),
        ],
        out_specs=pl.BlockSpec((_BQ, f), lambda i: (i, 0)),
        out_shape=jax.ShapeDtypeStruct((m, f), jnp.float32),
        compiler_params=pltpu.CompilerParams(
            dimension_semantics=("parallel",)),
    )(bias, pos_y, pos_x.T, x)


def kernel(x, pos_x, pos_y, k):
    m = pos_y.shape[0]
    n = pos_x.shape[0]
    f = x.shape[1]
    nblk = m // _BQ
    bias = (jnp.asarray(k, jnp.float32) - 3.0).reshape(1)

    order = jnp.argsort(pos_x[:, 0])
    keys_s = pos_x[order]
    x_s = jnp.concatenate(
        [x[order], jnp.ones((n, 1), jnp.float32)], axis=1)
    blk_lat = pos_y[:: _BQ, 0]
    counts = jnp.searchsorted(keys_s[:, 0], blk_lat).astype(jnp.int32)
    starts = jnp.clip(counts - _S // 2, 0, n - _S) & ~jnp.int32(7)

    fd = f // 3
    o0, o1, o2, flags = pl.pallas_call(
        _win_body,
        grid=(nblk,),
        in_specs=[
            pl.BlockSpec(memory_space=pltpu.SMEM),
            pl.BlockSpec(memory_space=pltpu.SMEM),
            pl.BlockSpec((2, _BQ), lambda i: (0, i)),
            pl.BlockSpec((n, 2), lambda i: (0, 0)),
            pl.BlockSpec((n, f + 1), lambda i: (0, 0)),
        ],
        out_specs=[
            pl.BlockSpec((_BQ, fd), lambda i: (i, 0)),
            pl.BlockSpec((_BQ, fd), lambda i: (i, 0)),
            pl.BlockSpec((_BQ, fd), lambda i: (i, 0)),
            pl.BlockSpec((1, 1, _BQ), lambda i: (i, 0, 0)),
        ],
        out_shape=[
            jax.ShapeDtypeStruct((m, fd), jnp.float32),
            jax.ShapeDtypeStruct((m, fd), jnp.float32),
            jax.ShapeDtypeStruct((m, fd), jnp.float32),
            jax.ShapeDtypeStruct((nblk, 1, _BQ), jnp.float32),
        ],
        compiler_params=pltpu.CompilerParams(
            dimension_semantics=("arbitrary",)),
    )(bias, starts, pos_y.T, keys_s, x_s)

    all_ok = jnp.all(flags > 0.5)
    return lax.cond(
        all_ok,
        lambda: jnp.stack([o0, o1, o2], axis=0),
        lambda: _full_kernel(bias, x, pos_x, pos_y)
        .reshape(m, 3, f // 3).transpose(1, 0, 2))


# speedup vs baseline: 1.1480x; 1.1480x over previous
import jax
import jax.numpy as jnp
from jax import lax
from jax.experimental import pallas as pl
from jax.experimental.pallas import tpu as pltpu

_N = 8192
_M = 65536
_F = 21
_BQ = 512
_S = 1024


def _win_body(bias_ref, start_ref, posyt_ref, keys_ref, x_ref,
              out_ref, flag_ref):
    start = start_ref[pl.program_id(0)]
    bias = bias_ref[0]
    big = jnp.float32(jnp.inf)
    qlat = posyt_ref[0:1, :]
    qlon = posyt_ref[1:2, :]
    kl = keys_ref[pl.ds(start, _S), 0:1]
    kn = keys_ref[pl.ds(start, _S), 1:2]
    dlat = kl - qlat
    dlon = kn - qlon
    d2 = dlat * dlat + dlon * dlon

    v1 = jnp.min(d2, axis=0, keepdims=True)
    v2 = jnp.min(jnp.where(d2 > v1, d2, big), axis=0, keepdims=True)
    v3 = jnp.min(jnp.where(d2 > v2, d2, big), axis=0, keepdims=True)
    w_mat = jnp.where(d2 <= v3,
                      1.0 / jnp.maximum(d2 + bias, 1e-16), 0.0)

    num = jax.lax.dot_general(
        w_mat, x_ref[pl.ds(start, _S), :],
        dimension_numbers=(((0,), (0,)), ((), ())),
        preferred_element_type=jnp.float32,
        precision=jax.lax.Precision.DEFAULT)
    rden = 1.0 / num[:, _F:_F + 1]
    out_ref[...] = num[:, 0:_F] * rden

    lat_first = jnp.min(keys_ref[pl.ds(start, 8), 0:1], axis=0,
                        keepdims=True)
    lat_last = jnp.max(keys_ref[pl.ds(start + (_S - 8), 8), 0:1], axis=0,
                       keepdims=True)
    dl = jnp.maximum(qlat - lat_first, 0.0)
    dr = jnp.maximum(lat_last - qlat, 0.0)
    bl = jnp.where(start == 0, big, dl * dl)
    br = jnp.where(start + _S == keys_ref.shape[0], big, dr * dr)
    ok = v3 < jnp.minimum(bl, br)
    flag_ref[...] = ok.astype(jnp.float32).reshape(1, 1, _BQ)


def _full_body(bias_ref, posy_ref, keys_ref, x_ref, out_ref):
    qlat = posy_ref[:, 0:1]
    qlon = posy_ref[:, 1:2]
    klat = keys_ref[0:1, :]
    klon = keys_ref[1:2, :]
    dlat = qlat - klat
    dlon = qlon - klon
    d2 = dlat * dlat + dlon * dlon

    bias = bias_ref[0]
    big = jnp.float32(jnp.inf)
    v1 = jnp.min(d2, axis=1, keepdims=True)
    v2 = jnp.min(jnp.where(d2 > v1, d2, big), axis=1, keepdims=True)
    v3 = jnp.min(jnp.where(d2 > v2, d2, big), axis=1, keepdims=True)
    w_mat = jnp.where(d2 <= v3,
                      1.0 / jnp.maximum(d2 + bias, 1e-16), 0.0)
    den = (1.0 / jnp.maximum(v1 + bias, 1e-16)
           + 1.0 / jnp.maximum(v2 + bias, 1e-16)
           + 1.0 / jnp.maximum(v3 + bias, 1e-16))
    num = jax.lax.dot_general(
        w_mat, x_ref[...],
        dimension_numbers=(((1,), (0,)), ((), ())),
        preferred_element_type=jnp.float32,
        precision=jax.lax.Precision.DEFAULT)
    out_ref[...] = num * (1.0 / den)


def _full_kernel(bias, x, pos_x, pos_y):
    m, n, f = pos_y.shape[0], pos_x.shape[0], x.shape[1]
    return pl.pallas_call(
        _full_body,
        grid=(m // _BQ,),
        in_specs=[
            pl.BlockSpec(memory_space=pltpu.SMEM),
            pl.BlockSpec((_BQ, 2), lambda i: (i, 0)),
            pl.BlockSpec((2, n), lambda i: (0, 0)),
            pl.BlockSpec((n, f), lambda i: (0, 0)),
        ],
        out_specs=pl.BlockSpec((_BQ, f), lambda i: (i, 0)),
        out_shape=jax.ShapeDtypeStruct((m, f), jnp.float32),
        compiler_params=pltpu.CompilerParams(
            dimension_semantics=("parallel",)),
    )(bias, pos_y, pos_x.T, x)


def kernel(x, pos_x, pos_y, k):
    m = pos_y.shape[0]
    n = pos_x.shape[0]
    f = x.shape[1]
    nblk = m // _BQ
    bias = (jnp.asarray(k, jnp.float32) - 3.0).reshape(1)

    order = jnp.argsort(pos_x[:, 0])
    keys_s = pos_x[order]
    x_s = jnp.concatenate(
        [x[order], jnp.ones((n, 1), jnp.float32)], axis=1)
    blk_lat = pos_y[:: _BQ, 0]
    counts = jnp.searchsorted(keys_s[:, 0], blk_lat).astype(jnp.int32)
    starts = jnp.clip(counts - _S // 2, 0, n - _S) & ~jnp.int32(7)

    out, flags = pl.pallas_call(
        _win_body,
        grid=(nblk,),
        in_specs=[
            pl.BlockSpec(memory_space=pltpu.SMEM),
            pl.BlockSpec(memory_space=pltpu.SMEM),
            pl.BlockSpec((2, _BQ), lambda i: (0, i)),
            pl.BlockSpec((n, 2), lambda i: (0, 0)),
            pl.BlockSpec((n, f + 1), lambda i: (0, 0)),
        ],
        out_specs=[
            pl.BlockSpec((_BQ, f), lambda i: (i, 0)),
            pl.BlockSpec((1, 1, _BQ), lambda i: (i, 0, 0)),
        ],
        out_shape=[
            jax.ShapeDtypeStruct((m, f), jnp.float32),
            jax.ShapeDtypeStruct((nblk, 1, _BQ), jnp.float32),
        ],
        compiler_params=pltpu.CompilerParams(
            dimension_semantics=("arbitrary",)),
    )(bias, starts, pos_y.T, keys_s, x_s)

    all_ok = jnp.all(flags > 0.5)
    out = lax.cond(all_ok,
                   lambda: out,
                   lambda: _full_kernel(bias, x, pos_x, pos_y))
    return out.reshape(m, 3, f // 3).transpose(1, 0, 2)


# speedup vs baseline: 1.1949x; 1.0409x over previous
import jax
import jax.numpy as jnp
from jax import lax
from jax.experimental import pallas as pl
from jax.experimental.pallas import tpu as pltpu

_N = 8192
_M = 65536
_F = 21
_BQ = 512
_S = 1024


def _win_body(bias_ref, start_ref, posyt_ref, keys_ref, x_ref,
              out_ref, flag_ref):
    start = start_ref[pl.program_id(0)]
    bias = bias_ref[0]
    big = jnp.float32(jnp.inf)
    qlat = posyt_ref[0:1, :]
    qlon = posyt_ref[1:2, :]
    kl = keys_ref[pl.ds(start, _S), 0:1]
    kn = keys_ref[pl.ds(start, _S), 1:2]
    dlat = kl - qlat
    dlon = kn - qlon
    d2 = dlat * dlat + dlon * dlon

    v1 = jnp.min(d2, axis=0, keepdims=True)
    v2 = jnp.min(jnp.where(d2 > v1, d2, big), axis=0, keepdims=True)
    v3 = jnp.min(jnp.where(d2 > v2, d2, big), axis=0, keepdims=True)
    w1 = 1.0 / jnp.maximum(v1 + bias, 1e-16)
    w2 = 1.0 / jnp.maximum(v2 + bias, 1e-16)
    w3 = 1.0 / jnp.maximum(v3 + bias, 1e-16)
    rden = 1.0 / (w1 + w2 + w3)
    w_mat = jnp.where(d2 <= v3,
                      rden / jnp.maximum(d2 + bias, 1e-16), 0.0)

    out_ref[...] = jax.lax.dot_general(
        w_mat, x_ref[pl.ds(start, _S), :],
        dimension_numbers=(((0,), (0,)), ((), ())),
        preferred_element_type=jnp.float32,
        precision=jax.lax.Precision.DEFAULT)

    lat_first = jnp.min(keys_ref[pl.ds(start, 8), 0:1], axis=0,
                        keepdims=True)
    lat_last = jnp.max(keys_ref[pl.ds(start + (_S - 8), 8), 0:1], axis=0,
                       keepdims=True)
    dl = jnp.maximum(qlat - lat_first, 0.0)
    dr = jnp.maximum(lat_last - qlat, 0.0)
    bl = jnp.where(start == 0, big, dl * dl)
    br = jnp.where(start + _S == keys_ref.shape[0], big, dr * dr)
    ok = v3 < jnp.minimum(bl, br)
    flag_ref[...] = ok.astype(jnp.float32).reshape(1, 1, _BQ)


def _full_body(bias_ref, posy_ref, keys_ref, x_ref, out_ref):
    qlat = posy_ref[:, 0:1]
    qlon = posy_ref[:, 1:2]
    klat = keys_ref[0:1, :]
    klon = keys_ref[1:2, :]
    dlat = qlat - klat
    dlon = qlon - klon
    d2 = dlat * dlat + dlon * dlon

    bias = bias_ref[0]
    big = jnp.float32(jnp.inf)
    v1 = jnp.min(d2, axis=1, keepdims=True)
    v2 = jnp.min(jnp.where(d2 > v1, d2, big), axis=1, keepdims=True)
    v3 = jnp.min(jnp.where(d2 > v2, d2, big), axis=1, keepdims=True)
    w_mat = jnp.where(d2 <= v3,
                      1.0 / jnp.maximum(d2 + bias, 1e-16), 0.0)
    den = (1.0 / jnp.maximum(v1 + bias, 1e-16)
           + 1.0 / jnp.maximum(v2 + bias, 1e-16)
           + 1.0 / jnp.maximum(v3 + bias, 1e-16))
    num = jax.lax.dot_general(
        w_mat, x_ref[...],
        dimension_numbers=(((1,), (0,)), ((), ())),
        preferred_element_type=jnp.float32,
        precision=jax.lax.Precision.DEFAULT)
    out_ref[...] = num * (1.0 / den)


def _full_kernel(bias, x, pos_x, pos_y):
    m, n, f = pos_y.shape[0], pos_x.shape[0], x.shape[1]
    return pl.pallas_call(
        _full_body,
        grid=(m // _BQ,),
        in_specs=[
            pl.BlockSpec(memory_space=pltpu.SMEM),
            pl.BlockSpec((_BQ, 2), lambda i: (i, 0)),
            pl.BlockSpec((2, n), lambda i: (0, 0)),
            pl.BlockSpec((n, f), lambda i: (0, 0)),
        ],
        out_specs=pl.BlockSpec((_BQ, f), lambda i: (i, 0)),
        out_shape=jax.ShapeDtypeStruct((m, f), jnp.float32),
        compiler_params=pltpu.CompilerParams(
            dimension_semantics=("parallel",)),
    )(bias, pos_y, pos_x.T, x)


def kernel(x, pos_x, pos_y, k):
    m = pos_y.shape[0]
    n = pos_x.shape[0]
    f = x.shape[1]
    nblk = m // _BQ
    bias = (jnp.asarray(k, jnp.float32) - 3.0).reshape(1)

    order = jnp.argsort(pos_x[:, 0])
    keys_s = pos_x[order]
    x_s = x[order]
    blk_lat = pos_y[:: _BQ, 0]
    counts = jnp.searchsorted(keys_s[:, 0], blk_lat).astype(jnp.int32)
    starts = jnp.clip(counts - _S // 2, 0, n - _S) & ~jnp.int32(7)

    out, flags = pl.pallas_call(
        _win_body,
        grid=(nblk,),
        in_specs=[
            pl.BlockSpec(memory_space=pltpu.SMEM),
            pl.BlockSpec(memory_space=pltpu.SMEM),
            pl.BlockSpec((2, _BQ), lambda i: (0, i)),
            pl.BlockSpec((n, 2), lambda i: (0, 0)),
            pl.BlockSpec((n, f), lambda i: (0, 0)),
        ],
        out_specs=[
            pl.BlockSpec((_BQ, f), lambda i: (i, 0)),
            pl.BlockSpec((1, 1, _BQ), lambda i: (i, 0, 0)),
        ],
        out_shape=[
            jax.ShapeDtypeStruct((m, f), jnp.float32),
            jax.ShapeDtypeStruct((nblk, 1, _BQ), jnp.float32),
        ],
        compiler_params=pltpu.CompilerParams(
            dimension_semantics=("arbitrary",)),
    )(bias, starts, pos_y.T, keys_s, x_s)

    all_ok = jnp.all(flags > 0.5)
    out = lax.cond(all_ok,
                   lambda: out,
                   lambda: _full_kernel(bias, x, pos_x, pos_y))
    return out.reshape(m, 3, f // 3).transpose(1, 0, 2)
